# baseline (speedup 1.0000x reference)
import functools

import jax
import jax.numpy as jnp
from jax import lax
from jax.experimental import pallas as pl
from jax.experimental.pallas import tpu as pltpu
from jax.experimental.pallas import tpu_sc as plsc

N_NODES = 50000
NODES_PAD = 50176
TRASH = 50048
ACC_ROWS = NODES_PAD
ROWS_PER_TILE = ACC_ROWS // 16
ZCH = 112
CHUNK = 128
E_PAD = 802816
N_CHUNKS = E_PAD // CHUNK
NCORES = 2
NSUBS = 16
BLK = 512
GRID = NODES_PAD // BLK

LAYER_DIMS = [(16, 16, 5), (16, 32, 4), (32, 64, 3), (64, 152, 2)]


def _sc_mesh():
    return plsc.VectorSubcoreMesh(core_axis_name="c", subcore_axis_name="s")


def _fill_rows(ref, value, rows, width):
    vec = jnp.full((16,), value, jnp.float32)

    def body(i, carry):
        for j in range(width // 16):
            ref[i, pl.ds(j * 16, 16)] = vec
        return carry

    lax.fori_loop(0, rows, body, 0)


def _zero_acc(acc, zbuf, sid):
    def body(t, carry):
        pltpu.sync_copy(
            zbuf, acc.at[pl.ds(sid * ROWS_PER_TILE + t * ZCH, ZCH)])
        return carry

    lax.fori_loop(0, ROWS_PER_TILE // ZCH, body, 0)


WB = 16


@functools.lru_cache(maxsize=None)
def _make_spmv(feature_split, slab0, nslabs):
    if feature_split:
        chunks_per_tile = N_CHUNKS // NSUBS
        group = 4
    else:
        chunks_per_tile = N_CHUNKS // (NCORES * NSUBS)
        group = 2
    n_groups = chunks_per_tile // group
    assert group * n_groups == chunks_per_tile and n_groups % 2 == 0

    @functools.partial(
        pl.kernel,
        out_type=jax.ShapeDtypeStruct((NCORES, ACC_ROWS, WB), jnp.float32),
        mesh=_sc_mesh(),
        compiler_params=pltpu.CompilerParams(use_tc_tiling_on_sc=False),
        scratch_types=[
            pltpu.VMEM_SHARED((ACC_ROWS, WB), jnp.float32),
            pltpu.VMEM_SHARED((NODES_PAD, WB), jnp.float32),
            pltpu.VMEM((ZCH, WB), jnp.float32),
            pltpu.VMEM((2, group, CHUNK), jnp.int32),
            pltpu.VMEM((2, group, CHUNK), jnp.int32),
            pltpu.VMEM((2, group, CHUNK, WB), jnp.float32),
            pltpu.SemaphoreType.DMA,
            pltpu.SemaphoreType.DMA,
        ],
    )
    def spmv(s_hbm, gidx_hbm, sidx_hbm, out_hbm, acc, tab, zbuf, gi, si, data,
             sem, ssem):
        cid = lax.axis_index("c")
        sid = lax.axis_index("s")
        _fill_rows(zbuf, 0.0, ZCH, WB)
        _zero_acc(acc, zbuf, sid)
        if feature_split:
            slab = (slab0 + cid) * NODES_PAD
        else:
            slab = 0
        pltpu.sync_copy(
            s_hbm.at[pl.ds(slab + sid * ROWS_PER_TILE, ROWS_PER_TILE)],
            tab.at[pl.ds(sid * ROWS_PER_TILE, ROWS_PER_TILE)])
        plsc.subcore_barrier()

        if feature_split:
            chunk0 = sid * chunks_per_tile
        else:
            chunk0 = (cid * NSUBS + sid) * chunks_per_tile

        def load_idx(slot, grp):
            base = chunk0 + grp * group
            pltpu.sync_copy(gidx_hbm.at[pl.ds(base, group)], gi.at[slot])
            pltpu.sync_copy(sidx_hbm.at[pl.ds(base, group)], si.at[slot])

        def fire(slot):
            for g in range(group):
                pltpu.async_copy(tab.at[gi.at[slot, g]],
                                 data.at[slot, g], sem)

        def drain_gather(slot):
            for g in range(group):
                pltpu.make_async_copy(tab.at[gi.at[slot, g]],
                                      data.at[slot, g], sem).wait()

        def fire_scatter(slot):
            for g in range(group):
                pltpu.async_copy(data.at[slot, g], acc.at[si.at[slot, g]],
                                 ssem, add=True)

        def wait_scatter(slot):
            for g in range(group):
                pltpu.make_async_copy(data.at[slot, g],
                                      acc.at[si.at[slot, g]], ssem).wait()

        load_idx(0, 0)
        fire(0)

        def body(o, carry):
            load_idx(1, 2 * o + 1)
            fire(1)
            drain_gather(0)
            fire_scatter(0)
            load_idx(0, lax.rem(2 * o + 2, n_groups))
            wait_scatter(0)
            fire(0)
            drain_gather(1)
            fire_scatter(1)
            wait_scatter(1)
            return carry

        lax.fori_loop(0, n_groups // 2, body, 0)
        for g in range(group):
            pltpu.make_async_copy(tab.at[gi.at[0, g]],
                                  data.at[0, g], sem).wait()
        plsc.subcore_barrier()
        pltpu.sync_copy(
            acc.at[pl.ds(sid * ROWS_PER_TILE, ROWS_PER_TILE)],
            out_hbm.at[cid, pl.ds(sid * ROWS_PER_TILE, ROWS_PER_TILE)])

    return spmv


@functools.lru_cache(maxsize=None)
def _make_degree():
    wb = 16
    chunks_per_tile = N_CHUNKS // (NCORES * NSUBS)
    group = 7
    n_groups = chunks_per_tile // group

    @functools.partial(
        pl.kernel,
        out_type=jax.ShapeDtypeStruct((NCORES, ACC_ROWS, wb), jnp.float32),
        mesh=_sc_mesh(),
        compiler_params=pltpu.CompilerParams(use_tc_tiling_on_sc=False),
        scratch_types=[
            pltpu.VMEM_SHARED((ACC_ROWS, wb), jnp.float32),
            pltpu.VMEM((ZCH, wb), jnp.float32),
            pltpu.VMEM((CHUNK, wb), jnp.float32),
            pltpu.VMEM((2, group, CHUNK), jnp.int32),
            pltpu.SemaphoreType.DMA,
        ],
    )
    def degree(sidx_hbm, out_hbm, acc, zbuf, obuf, si, sem):
        cid = lax.axis_index("c")
        sid = lax.axis_index("s")
        _fill_rows(zbuf, 0.0, ZCH, wb)
        _fill_rows(obuf, 1.0, CHUNK, wb)
        _zero_acc(acc, zbuf, sid)
        plsc.subcore_barrier()
        chunk0 = (cid * NSUBS + sid) * chunks_per_tile

        def load_idx(slot, grp):
            base = chunk0 + grp * group
            pltpu.sync_copy(sidx_hbm.at[pl.ds(base, group)], si.at[slot])

        def scatter(slot):
            for g in range(group):
                pltpu.async_copy(obuf, acc.at[si.at[slot, g]], sem, add=True)
            for g in range(group):
                pltpu.make_async_copy(obuf, acc.at[si.at[slot, g]],
                                      sem).wait()

        load_idx(0, 0)

        def body(o, carry):
            load_idx(1, 2 * o + 1)
            scatter(0)
            load_idx(0, lax.rem(2 * o + 2, n_groups))
            scatter(1)
            return carry

        lax.fori_loop(0, n_groups // 2, body, 0)
        plsc.subcore_barrier()
        pltpu.sync_copy(
            acc.at[pl.ds(sid * ROWS_PER_TILE, ROWS_PER_TILE)],
            out_hbm.at[cid, pl.ds(sid * ROWS_PER_TILE, ROWS_PER_TILE)])

    return degree


def _prep_body(deg_ref, x_ref, dis_ref, s0_ref):
    d = deg_ref[0, :, 0:1] + deg_ref[1, :, 0:1]
    dis = jnp.where(d > 0, lax.rsqrt(d), 0.0)
    dis_ref[...] = jnp.broadcast_to(dis, (BLK, 64))
    s0_ref[...] = x_ref[...] * dis


@functools.lru_cache(maxsize=None)
def _make_prep():
    return pl.pallas_call(
        _prep_body,
        grid=(GRID,),
        in_specs=[
            pl.BlockSpec((2, BLK, 16), lambda i: (0, i, 0)),
            pl.BlockSpec((BLK, 16), lambda i: (i, 0)),
        ],
        out_specs=[
            pl.BlockSpec((BLK, 64), lambda i: (i, 0)),
            pl.BlockSpec((BLK, 16), lambda i: (i, 0)),
        ],
        out_shape=[
            jax.ShapeDtypeStruct((NODES_PAD, 64), jnp.float32),
            jax.ShapeDtypeStruct((NODES_PAD, 16), jnp.float32),
        ],
    )


@functools.lru_cache(maxsize=None)
def _make_combine(naccs, ncat, first, want_s, split_s):
    wo = 16 * max(ncat, 1)

    def body(*refs):
        accs = refs[:naccs]
        dis_ref = refs[naccs]
        idx = naccs + 1
        if not first:
            txp_ref = refs[idx]
            idx += 1
        out_refs = refs[idx:]
        parts = []
        for a_ref in accs:
            parts += [a_ref[0], a_ref[1]]
        if ncat == 1:
            a = parts[0] + parts[1]
        else:
            a = jnp.concatenate(parts, axis=-1)
        dis = dis_ref[:, :wo]
        if first:
            tx = -(dis * a)
        else:
            tx = -2.0 * (dis * a) - txp_ref[...]
        out_refs[0][...] = tx
        if want_s:
            sv = dis * tx
            if split_s:
                for c in range(split_s):
                    out_refs[1][c] = sv[:, c * 16:(c + 1) * 16]
            else:
                out_refs[1][...] = sv

    in_specs = [pl.BlockSpec((2, BLK, 16), lambda i: (0, i, 0))
                for _ in range(naccs)]
    in_specs.append(pl.BlockSpec((BLK, 64), lambda i: (i, 0)))
    if not first:
        in_specs.append(pl.BlockSpec((BLK, wo), lambda i: (i, 0)))
    out_specs = [pl.BlockSpec((BLK, wo), lambda i: (i, 0))]
    out_shape = [jax.ShapeDtypeStruct((NODES_PAD, wo), jnp.float32)]
    if want_s:
        if split_s:
            out_specs.append(
                pl.BlockSpec((split_s, BLK, 16), lambda i: (0, i, 0)))
            out_shape.append(
                jax.ShapeDtypeStruct((split_s, NODES_PAD, 16), jnp.float32))
        else:
            out_specs.append(pl.BlockSpec((BLK, wo), lambda i: (i, 0)))
            out_shape.append(
                jax.ShapeDtypeStruct((NODES_PAD, wo), jnp.float32))
    return pl.pallas_call(
        body, grid=(GRID,), in_specs=in_specs, out_specs=out_specs,
        out_shape=out_shape)


@functools.lru_cache(maxsize=None)
def _make_gru(nk, cin, cout, nsplit):
    def body(*refs):
        txs = refs[:nk]
        wz_ref, wh_ref, bz_ref, bh_ref = refs[nk:nk + 4]
        rest = refs[nk + 4:]
        az = jnp.zeros((BLK, cout), jnp.float32)
        ah = jnp.zeros((BLK, cout), jnp.float32)
        for k in range(nk):
            xk = txs[k][...]
            az = az + jnp.dot(xk, wz_ref[k], preferred_element_type=jnp.float32)
            ah = ah + jnp.dot(xk, wh_ref[k], preferred_element_type=jnp.float32)
        z = jax.nn.sigmoid(az + bz_ref[...])
        ht = jnp.tanh(ah + bh_ref[...])
        h = jax.nn.relu((1.0 - z) * ht)
        if nsplit < 0:
            wl_ref, bl_ref, out_ref = rest
            logits = jnp.dot(h, wl_ref[...],
                             preferred_element_type=jnp.float32) + bl_ref[...]
            out_ref[...] = jax.nn.softmax(logits, axis=-1)
        elif nsplit == 0:
            dis_ref, h_ref, s_ref = rest
            h_ref[...] = h
            s_ref[...] = h * dis_ref[:, :cout]
        else:
            dis_ref, h_ref, s2_ref = rest
            h_ref[...] = h
            sv = h * dis_ref[:, :cout]
            for c in range(nsplit):
                s2_ref[c] = sv[:, c * 16:(c + 1) * 16]

    in_specs = [pl.BlockSpec((BLK, cin), lambda i: (i, 0)) for _ in range(nk)]
    in_specs += [
        pl.BlockSpec((nk, cin, cout), lambda i: (0, 0, 0)),
        pl.BlockSpec((nk, cin, cout), lambda i: (0, 0, 0)),
        pl.BlockSpec((1, cout), lambda i: (0, 0)),
        pl.BlockSpec((1, cout), lambda i: (0, 0)),
    ]
    if nsplit < 0:
        in_specs += [
            pl.BlockSpec((cout, 2), lambda i: (0, 0)),
            pl.BlockSpec((1, 2), lambda i: (0, 0)),
        ]
        out_specs = [pl.BlockSpec((BLK, 2), lambda i: (i, 0))]
        out_shape = [jax.ShapeDtypeStruct((NODES_PAD, 2), jnp.float32)]
    elif nsplit == 0:
        in_specs.append(pl.BlockSpec((BLK, 64), lambda i: (i, 0)))
        out_specs = [
            pl.BlockSpec((BLK, cout), lambda i: (i, 0)),
            pl.BlockSpec((BLK, cout), lambda i: (i, 0)),
        ]
        out_shape = [
            jax.ShapeDtypeStruct((NODES_PAD, cout), jnp.float32),
            jax.ShapeDtypeStruct((NODES_PAD, cout), jnp.float32),
        ]
    else:
        in_specs.append(pl.BlockSpec((BLK, 64), lambda i: (i, 0)))
        out_specs = [
            pl.BlockSpec((BLK, cout), lambda i: (i, 0)),
            pl.BlockSpec((nsplit, BLK, 16), lambda i: (0, i, 0)),
        ]
        out_shape = [
            jax.ShapeDtypeStruct((NODES_PAD, cout), jnp.float32),
            jax.ShapeDtypeStruct((nsplit, NODES_PAD, 16), jnp.float32),
        ]
    return pl.pallas_call(
        body, grid=(GRID,), in_specs=in_specs, out_specs=out_specs,
        out_shape=out_shape)


def _pad_w(w, cin):
    ci = w.shape[1]
    if ci == cin:
        return w
    return jnp.pad(w, ((0, 0), (0, cin - ci), (0, 0)))


def kernel(x, edge_index, params):
    row, col = edge_index[0], edge_index[1]
    npad = E_PAD - row.shape[0]
    trash = jnp.full((npad,), TRASH, jnp.int32)
    row2d = jnp.concatenate([row, trash]).reshape(N_CHUNKS, CHUNK)
    col2d = jnp.concatenate([col, trash]).reshape(N_CHUNKS, CHUNK)
    x16 = jnp.pad(x, ((0, NODES_PAD - x.shape[0]), (0, 16 - x.shape[1])))

    deg_parts = _make_degree()(row2d)
    dis64, s = _make_prep()(deg_parts, x16)

    def weights(lp, cin, cout):
        wz = _pad_w(lp["xz"][0], cin)
        wh = _pad_w(lp["xh"][0], cin)
        bz = (lp["xz"][1] + lp["hz"][1]).reshape(1, cout)
        bh = (lp["xh"][1] + lp["hh"][1]).reshape(1, cout)
        return wz, wh, bz, bh

    h = x16
    for li in (0, 1):
        cin, cout, K = LAYER_DIMS[li]
        txs = [h]
        for k in range(1, K):
            first = k == 1
            want_s = k < K - 1
            acc = _make_spmv(False, 0, 1)(s, row2d, col2d)
            args = (acc, dis64) if first else (acc, dis64, txs[k - 2])
            outs = _make_combine(1, 1, first, want_s, 0)(*args)
            txs.append(outs[0])
            if want_s:
                s = outs[1]
        nsplit = 0 if li == 0 else 2
        hh, s_next = _make_gru(K, cin, cout, nsplit)(
            *txs, *weights(params["layers"][li], cin, cout), dis64)
        h = hh
        s = s_next if nsplit == 0 else s_next.reshape(2 * NODES_PAD, 16)

    cin, cout, K = LAYER_DIMS[2]
    acc = _make_spmv(True, 0, 2)(s, row2d, col2d)
    tx1, s2 = _make_combine(1, 2, True, True, 2)(acc, dis64)
    acc = _make_spmv(True, 0, 2)(s2.reshape(2 * NODES_PAD, 16), row2d, col2d)
    (tx2,) = _make_combine(1, 2, False, False, 0)(acc, dis64, h)
    h3, s4 = _make_gru(K, cin, cout, 4)(
        h, tx1, tx2, *weights(params["layers"][2], cin, cout), dis64)

    cin, cout, K = LAYER_DIMS[3]
    s4f = s4.reshape(4 * NODES_PAD, 16)
    acc_a = _make_spmv(True, 0, 4)(s4f, row2d, col2d)
    acc_b = _make_spmv(True, 2, 4)(s4f, row2d, col2d)
    (tx,) = _make_combine(2, 4, True, False, 0)(acc_a, acc_b, dis64)
    wl, bl = params["linear"]
    (out,) = _make_gru(K, cin, cout, -1)(
        h3, tx, *weights(params["layers"][3], cin, cout), wl,
        bl.reshape(1, 2))
    return out[:N_NODES]


# speedup vs baseline: 1.0782x; 1.0782x over previous
import functools

import jax
import jax.numpy as jnp
from jax import lax
from jax.experimental import pallas as pl
from jax.experimental.pallas import tpu as pltpu
from jax.experimental.pallas import tpu_sc as plsc

N_NODES = 50000
NODES_PAD = 50176
TRASH = 50048
ACC_ROWS = NODES_PAD
ROWS_PER_TILE = ACC_ROWS // 16
ZCH = 112
CHUNK = 512
E_PAD = 819200
N_CHUNKS = E_PAD // CHUNK
NCORES = 2
NSUBS = 16
BLK = 512
GRID = NODES_PAD // BLK

LAYER_DIMS = [(16, 16, 5), (16, 32, 4), (32, 64, 3), (64, 152, 2)]


def _sc_mesh():
    return plsc.VectorSubcoreMesh(core_axis_name="c", subcore_axis_name="s")


def _fill_rows(ref, value, rows, width):
    vec = jnp.full((16,), value, jnp.float32)

    def body(i, carry):
        for j in range(width // 16):
            ref[i, pl.ds(j * 16, 16)] = vec
        return carry

    lax.fori_loop(0, rows, body, 0)


def _zero_acc(acc, zbuf, sid):
    def body(t, carry):
        pltpu.sync_copy(
            zbuf, acc.at[pl.ds(sid * ROWS_PER_TILE + t * ZCH, ZCH)])
        return carry

    lax.fori_loop(0, ROWS_PER_TILE // ZCH, body, 0)


WB = 16


@functools.lru_cache(maxsize=None)
def _make_spmv(feature_split, slab0, nslabs):
    if feature_split:
        chunks_per_tile = N_CHUNKS // NSUBS
    else:
        chunks_per_tile = N_CHUNKS // (NCORES * NSUBS)
    group = 1
    n_groups = chunks_per_tile // group
    assert group * n_groups == chunks_per_tile and n_groups % 2 == 0

    @functools.partial(
        pl.kernel,
        out_type=jax.ShapeDtypeStruct((NCORES, ACC_ROWS, WB), jnp.float32),
        mesh=_sc_mesh(),
        compiler_params=pltpu.CompilerParams(use_tc_tiling_on_sc=False),
        scratch_types=[
            pltpu.VMEM_SHARED((ACC_ROWS, WB), jnp.float32),
            pltpu.VMEM_SHARED((NODES_PAD, WB), jnp.float32),
            pltpu.VMEM((ZCH, WB), jnp.float32),
            pltpu.VMEM((2, group, CHUNK), jnp.int32),
            pltpu.VMEM((2, group, CHUNK), jnp.int32),
            pltpu.VMEM((2, group, CHUNK, WB), jnp.float32),
            pltpu.SemaphoreType.DMA,
            pltpu.SemaphoreType.DMA,
        ],
    )
    def spmv(s_hbm, gidx_hbm, sidx_hbm, out_hbm, acc, tab, zbuf, gi, si, data,
             sem, ssem):
        cid = lax.axis_index("c")
        sid = lax.axis_index("s")
        _fill_rows(zbuf, 0.0, ZCH, WB)
        _zero_acc(acc, zbuf, sid)
        if feature_split:
            slab = (slab0 + cid) * NODES_PAD
        else:
            slab = 0
        pltpu.sync_copy(
            s_hbm.at[pl.ds(slab + sid * ROWS_PER_TILE, ROWS_PER_TILE)],
            tab.at[pl.ds(sid * ROWS_PER_TILE, ROWS_PER_TILE)])
        plsc.subcore_barrier()

        if feature_split:
            chunk0 = sid * chunks_per_tile
        else:
            chunk0 = (cid * NSUBS + sid) * chunks_per_tile

        def load_idx(slot, grp):
            base = chunk0 + grp * group
            pltpu.sync_copy(gidx_hbm.at[pl.ds(base, group)], gi.at[slot])
            pltpu.sync_copy(sidx_hbm.at[pl.ds(base, group)], si.at[slot])

        def fire(slot):
            for g in range(group):
                pltpu.async_copy(tab.at[gi.at[slot, g]],
                                 data.at[slot, g], sem)

        def drain_gather(slot):
            for g in range(group):
                pltpu.make_async_copy(tab.at[gi.at[slot, g]],
                                      data.at[slot, g], sem).wait()

        def fire_scatter(slot):
            for g in range(group):
                pltpu.async_copy(data.at[slot, g], acc.at[si.at[slot, g]],
                                 ssem, add=True)

        def wait_scatter(slot):
            for g in range(group):
                pltpu.make_async_copy(data.at[slot, g],
                                      acc.at[si.at[slot, g]], ssem).wait()

        load_idx(0, 0)
        fire(0)

        def body(o, carry):
            load_idx(1, 2 * o + 1)
            fire(1)
            drain_gather(0)
            fire_scatter(0)
            load_idx(0, lax.rem(2 * o + 2, n_groups))
            wait_scatter(0)
            fire(0)
            drain_gather(1)
            fire_scatter(1)
            wait_scatter(1)
            return carry

        lax.fori_loop(0, n_groups // 2, body, 0)
        for g in range(group):
            pltpu.make_async_copy(tab.at[gi.at[0, g]],
                                  data.at[0, g], sem).wait()
        plsc.subcore_barrier()
        pltpu.sync_copy(
            acc.at[pl.ds(sid * ROWS_PER_TILE, ROWS_PER_TILE)],
            out_hbm.at[cid, pl.ds(sid * ROWS_PER_TILE, ROWS_PER_TILE)])

    return spmv


@functools.lru_cache(maxsize=None)
def _make_degree():
    wb = 16
    chunks_per_tile = N_CHUNKS // (NCORES * NSUBS)
    group = 1
    n_groups = chunks_per_tile // group

    @functools.partial(
        pl.kernel,
        out_type=jax.ShapeDtypeStruct((NCORES, ACC_ROWS, wb), jnp.float32),
        mesh=_sc_mesh(),
        compiler_params=pltpu.CompilerParams(use_tc_tiling_on_sc=False),
        scratch_types=[
            pltpu.VMEM_SHARED((ACC_ROWS, wb), jnp.float32),
            pltpu.VMEM((ZCH, wb), jnp.float32),
            pltpu.VMEM((CHUNK, wb), jnp.float32),
            pltpu.VMEM((2, group, CHUNK), jnp.int32),
            pltpu.SemaphoreType.DMA,
        ],
    )
    def degree(sidx_hbm, out_hbm, acc, zbuf, obuf, si, sem):
        cid = lax.axis_index("c")
        sid = lax.axis_index("s")
        _fill_rows(zbuf, 0.0, ZCH, wb)
        _fill_rows(obuf, 1.0, CHUNK, wb)
        _zero_acc(acc, zbuf, sid)
        plsc.subcore_barrier()
        chunk0 = (cid * NSUBS + sid) * chunks_per_tile

        def load_idx(slot, grp):
            base = chunk0 + grp * group
            pltpu.sync_copy(sidx_hbm.at[pl.ds(base, group)], si.at[slot])

        def scatter(slot):
            for g in range(group):
                pltpu.async_copy(obuf, acc.at[si.at[slot, g]], sem, add=True)
            for g in range(group):
                pltpu.make_async_copy(obuf, acc.at[si.at[slot, g]],
                                      sem).wait()

        load_idx(0, 0)

        def body(o, carry):
            load_idx(1, 2 * o + 1)
            scatter(0)
            load_idx(0, lax.rem(2 * o + 2, n_groups))
            scatter(1)
            return carry

        lax.fori_loop(0, n_groups // 2, body, 0)
        plsc.subcore_barrier()
        pltpu.sync_copy(
            acc.at[pl.ds(sid * ROWS_PER_TILE, ROWS_PER_TILE)],
            out_hbm.at[cid, pl.ds(sid * ROWS_PER_TILE, ROWS_PER_TILE)])

    return degree


def _prep_body(deg_ref, x_ref, dis_ref, s0_ref):
    d = deg_ref[0, :, 0:1] + deg_ref[1, :, 0:1]
    dis = jnp.where(d > 0, lax.rsqrt(d), 0.0)
    dis_ref[...] = jnp.broadcast_to(dis, (BLK, 64))
    s0_ref[...] = x_ref[...] * dis


@functools.lru_cache(maxsize=None)
def _make_prep():
    return pl.pallas_call(
        _prep_body,
        grid=(GRID,),
        in_specs=[
            pl.BlockSpec((2, BLK, 16), lambda i: (0, i, 0)),
            pl.BlockSpec((BLK, 16), lambda i: (i, 0)),
        ],
        out_specs=[
            pl.BlockSpec((BLK, 64), lambda i: (i, 0)),
            pl.BlockSpec((BLK, 16), lambda i: (i, 0)),
        ],
        out_shape=[
            jax.ShapeDtypeStruct((NODES_PAD, 64), jnp.float32),
            jax.ShapeDtypeStruct((NODES_PAD, 16), jnp.float32),
        ],
    )


@functools.lru_cache(maxsize=None)
def _make_combine(naccs, ncat, first, want_s, split_s):
    wo = 16 * max(ncat, 1)

    def body(*refs):
        accs = refs[:naccs]
        dis_ref = refs[naccs]
        idx = naccs + 1
        if not first:
            txp_ref = refs[idx]
            idx += 1
        out_refs = refs[idx:]
        parts = []
        for a_ref in accs:
            parts += [a_ref[0], a_ref[1]]
        if ncat == 1:
            a = parts[0] + parts[1]
        else:
            a = jnp.concatenate(parts, axis=-1)
        dis = dis_ref[:, :wo]
        if first:
            tx = -(dis * a)
        else:
            tx = -2.0 * (dis * a) - txp_ref[...]
        out_refs[0][...] = tx
        if want_s:
            sv = dis * tx
            if split_s:
                for c in range(split_s):
                    out_refs[1][c] = sv[:, c * 16:(c + 1) * 16]
            else:
                out_refs[1][...] = sv

    in_specs = [pl.BlockSpec((2, BLK, 16), lambda i: (0, i, 0))
                for _ in range(naccs)]
    in_specs.append(pl.BlockSpec((BLK, 64), lambda i: (i, 0)))
    if not first:
        in_specs.append(pl.BlockSpec((BLK, wo), lambda i: (i, 0)))
    out_specs = [pl.BlockSpec((BLK, wo), lambda i: (i, 0))]
    out_shape = [jax.ShapeDtypeStruct((NODES_PAD, wo), jnp.float32)]
    if want_s:
        if split_s:
            out_specs.append(
                pl.BlockSpec((split_s, BLK, 16), lambda i: (0, i, 0)))
            out_shape.append(
                jax.ShapeDtypeStruct((split_s, NODES_PAD, 16), jnp.float32))
        else:
            out_specs.append(pl.BlockSpec((BLK, wo), lambda i: (i, 0)))
            out_shape.append(
                jax.ShapeDtypeStruct((NODES_PAD, wo), jnp.float32))
    return pl.pallas_call(
        body, grid=(GRID,), in_specs=in_specs, out_specs=out_specs,
        out_shape=out_shape)


@functools.lru_cache(maxsize=None)
def _make_gru(nk, cin, cout, nsplit):
    def body(*refs):
        txs = refs[:nk]
        wz_ref, wh_ref, bz_ref, bh_ref = refs[nk:nk + 4]
        rest = refs[nk + 4:]
        az = jnp.zeros((BLK, cout), jnp.float32)
        ah = jnp.zeros((BLK, cout), jnp.float32)
        for k in range(nk):
            xk = txs[k][...]
            az = az + jnp.dot(xk, wz_ref[k], preferred_element_type=jnp.float32)
            ah = ah + jnp.dot(xk, wh_ref[k], preferred_element_type=jnp.float32)
        z = jax.nn.sigmoid(az + bz_ref[...])
        ht = jnp.tanh(ah + bh_ref[...])
        h = jax.nn.relu((1.0 - z) * ht)
        if nsplit < 0:
            wl_ref, bl_ref, out_ref = rest
            logits = jnp.dot(h, wl_ref[...],
                             preferred_element_type=jnp.float32) + bl_ref[...]
            out_ref[...] = jax.nn.softmax(logits, axis=-1)
        elif nsplit == 0:
            dis_ref, h_ref, s_ref = rest
            h_ref[...] = h
            s_ref[...] = h * dis_ref[:, :cout]
        else:
            dis_ref, h_ref, s2_ref = rest
            h_ref[...] = h
            sv = h * dis_ref[:, :cout]
            for c in range(nsplit):
                s2_ref[c] = sv[:, c * 16:(c + 1) * 16]

    in_specs = [pl.BlockSpec((BLK, cin), lambda i: (i, 0)) for _ in range(nk)]
    in_specs += [
        pl.BlockSpec((nk, cin, cout), lambda i: (0, 0, 0)),
        pl.BlockSpec((nk, cin, cout), lambda i: (0, 0, 0)),
        pl.BlockSpec((1, cout), lambda i: (0, 0)),
        pl.BlockSpec((1, cout), lambda i: (0, 0)),
    ]
    if nsplit < 0:
        in_specs += [
            pl.BlockSpec((cout, 2), lambda i: (0, 0)),
            pl.BlockSpec((1, 2), lambda i: (0, 0)),
        ]
        out_specs = [pl.BlockSpec((BLK, 2), lambda i: (i, 0))]
        out_shape = [jax.ShapeDtypeStruct((NODES_PAD, 2), jnp.float32)]
    elif nsplit == 0:
        in_specs.append(pl.BlockSpec((BLK, 64), lambda i: (i, 0)))
        out_specs = [
            pl.BlockSpec((BLK, cout), lambda i: (i, 0)),
            pl.BlockSpec((BLK, cout), lambda i: (i, 0)),
        ]
        out_shape = [
            jax.ShapeDtypeStruct((NODES_PAD, cout), jnp.float32),
            jax.ShapeDtypeStruct((NODES_PAD, cout), jnp.float32),
        ]
    else:
        in_specs.append(pl.BlockSpec((BLK, 64), lambda i: (i, 0)))
        out_specs = [
            pl.BlockSpec((BLK, cout), lambda i: (i, 0)),
            pl.BlockSpec((nsplit, BLK, 16), lambda i: (0, i, 0)),
        ]
        out_shape = [
            jax.ShapeDtypeStruct((NODES_PAD, cout), jnp.float32),
            jax.ShapeDtypeStruct((nsplit, NODES_PAD, 16), jnp.float32),
        ]
    return pl.pallas_call(
        body, grid=(GRID,), in_specs=in_specs, out_specs=out_specs,
        out_shape=out_shape)


def _pad_w(w, cin):
    ci = w.shape[1]
    if ci == cin:
        return w
    return jnp.pad(w, ((0, 0), (0, cin - ci), (0, 0)))


def kernel(x, edge_index, params):
    row, col = edge_index[0], edge_index[1]
    npad = E_PAD - row.shape[0]
    trash = jnp.full((npad,), TRASH, jnp.int32)
    row2d = jnp.concatenate([row, trash]).reshape(N_CHUNKS, CHUNK)
    col2d = jnp.concatenate([col, trash]).reshape(N_CHUNKS, CHUNK)
    x16 = jnp.pad(x, ((0, NODES_PAD - x.shape[0]), (0, 16 - x.shape[1])))

    deg_parts = _make_degree()(row2d)
    dis64, s = _make_prep()(deg_parts, x16)

    def weights(lp, cin, cout):
        wz = _pad_w(lp["xz"][0], cin)
        wh = _pad_w(lp["xh"][0], cin)
        bz = (lp["xz"][1] + lp["hz"][1]).reshape(1, cout)
        bh = (lp["xh"][1] + lp["hh"][1]).reshape(1, cout)
        return wz, wh, bz, bh

    h = x16
    for li in (0, 1):
        cin, cout, K = LAYER_DIMS[li]
        txs = [h]
        for k in range(1, K):
            first = k == 1
            want_s = k < K - 1
            acc = _make_spmv(False, 0, 1)(s, row2d, col2d)
            args = (acc, dis64) if first else (acc, dis64, txs[k - 2])
            outs = _make_combine(1, 1, first, want_s, 0)(*args)
            txs.append(outs[0])
            if want_s:
                s = outs[1]
        nsplit = 0 if li == 0 else 2
        hh, s_next = _make_gru(K, cin, cout, nsplit)(
            *txs, *weights(params["layers"][li], cin, cout), dis64)
        h = hh
        s = s_next if nsplit == 0 else s_next.reshape(2 * NODES_PAD, 16)

    cin, cout, K = LAYER_DIMS[2]
    acc = _make_spmv(True, 0, 2)(s, row2d, col2d)
    tx1, s2 = _make_combine(1, 2, True, True, 2)(acc, dis64)
    acc = _make_spmv(True, 0, 2)(s2.reshape(2 * NODES_PAD, 16), row2d, col2d)
    (tx2,) = _make_combine(1, 2, False, False, 0)(acc, dis64, h)
    h3, s4 = _make_gru(K, cin, cout, 4)(
        h, tx1, tx2, *weights(params["layers"][2], cin, cout), dis64)

    cin, cout, K = LAYER_DIMS[3]
    s4f = s4.reshape(4 * NODES_PAD, 16)
    acc_a = _make_spmv(True, 0, 4)(s4f, row2d, col2d)
    acc_b = _make_spmv(True, 2, 4)(s4f, row2d, col2d)
    (tx,) = _make_combine(2, 4, True, False, 0)(acc_a, acc_b, dis64)
    wl, bl = params["linear"]
    (out,) = _make_gru(K, cin, cout, -1)(
        h3, tx, *weights(params["layers"][3], cin, cout), wl,
        bl.reshape(1, 2))
    return out[:N_NODES]
